# baseline (speedup 1.0000x reference)
import functools

import jax
import jax.numpy as jnp
from jax import lax
from jax.experimental import pallas as pl
from jax.experimental.pallas import tpu as pltpu

N = 50000
E = 800000
H = 64

_BN = 2000
_BE = 4000

_EPS = 1e-5


def _ln(h, g, beta):
    mu = jnp.mean(h, axis=-1, keepdims=True)
    var = jnp.mean((h - mu) ** 2, axis=-1, keepdims=True)
    return (h - mu) * lax.rsqrt(var + _EPS) * g + beta


def _dot(a, b):
    return jax.lax.dot_general(a, b, (((1,), (0,)), ((), ())),
                               preferred_element_type=jnp.float32)


def _enc_node_body(x_ref, phi_ref, w1a_ref, w1b_ref, b1_ref, w2_ref, b2_ref,
                   g_ref, beta_ref, out_ref):
    x = x_ref[...]
    phi = phi_ref[0, 0]
    load = jnp.where(x[:, 7:8] == 1.0, phi, 0.0)
    h = _dot(x, w1a_ref[...]) + load * w1b_ref[...] + b1_ref[...]
    h = jax.nn.relu(h)
    h = _dot(h, w2_ref[...]) + b2_ref[...]
    out_ref[...] = _ln(h, g_ref[...], beta_ref[...])


def _enc_edge_body(a_ref, w1_ref, b1_ref, w2_ref, b2_ref, g_ref, beta_ref,
                   out_ref):
    h = _dot(a_ref[...], w1_ref[...]) + b1_ref[...]
    h = jax.nn.relu(h)
    h = _dot(h, w2_ref[...]) + b2_ref[...]
    out_ref[...] = _ln(h, g_ref[...], beta_ref[...])


def _edge_body(xi_ref, xj_ref, eh_ref,
               a1_ref, a2_ref, a3_ref, eb1_ref, ew2_ref, eb2_ref, eg_ref,
               ebeta_ref, m1_ref, m2_ref, mb1_ref, mw2_ref, mb2_ref, mg_ref,
               mbeta_ref, eout_ref, msg_ref):
    xi = xi_ref[...]
    xj = xj_ref[...]
    eh = eh_ref[...]
    h = _dot(xi, a1_ref[...]) + _dot(xj, a2_ref[...]) + _dot(eh, a3_ref[...])
    h = jax.nn.relu(h + eb1_ref[...])
    e_upd = _ln(_dot(h, ew2_ref[...]) + eb2_ref[...], eg_ref[...], ebeta_ref[...])
    h2 = jax.nn.relu(_dot(xi, m1_ref[...]) + _dot(e_upd, m2_ref[...]) + mb1_ref[...])
    msg = _ln(_dot(h2, mw2_ref[...]) + mb2_ref[...], mg_ref[...], mbeta_ref[...])
    eout_ref[...] = e_upd + eh
    msg_ref[...] = msg


def _node_body(xh_ref, agg_ref, n1_ref, n2_ref, b1_ref, w2_ref, b2_ref,
               g_ref, beta_ref, out_ref):
    xh = xh_ref[...]
    h = jax.nn.relu(_dot(xh, n1_ref[...]) + _dot(agg_ref[...], n2_ref[...])
                    + b1_ref[...])
    h = _dot(h, w2_ref[...]) + b2_ref[...]
    out_ref[...] = _ln(h, g_ref[...], beta_ref[...]) + xh


def _dec_body(xh_ref, w1_ref, b1_ref, w2_ref, b2_ref, out_ref):
    h = jax.nn.relu(_dot(xh_ref[...], w1_ref[...]) + b1_ref[...])
    out_ref[...] = _dot(h, w2_ref[...]) + b2_ref[...]


def _row_spec(bs, width):
    return pl.BlockSpec((bs, width), lambda i: (i, 0))


def _const_spec(shape):
    nd = len(shape)
    return pl.BlockSpec(shape, lambda i: (0,) * nd)


def _call(body, grid, in_specs, out_specs, out_shape):
    return pl.pallas_call(
        body,
        grid=(grid,),
        in_specs=in_specs,
        out_specs=out_specs,
        out_shape=out_shape,
        compiler_params=pltpu.CompilerParams(
            dimension_semantics=("arbitrary",)),
    )


def kernel(x, edge_index, edge_attr, swelling_phi, enc_n_W1, enc_n_b1,
           enc_n_W2, enc_n_b2, enc_n_g, enc_n_beta, enc_e_W1, enc_e_b1,
           enc_e_W2, enc_e_b2, enc_e_g, enc_e_beta, pe_W1, pe_b1, pe_W2,
           pe_b2, pe_g, pe_beta, pm_W1, pm_b1, pm_W2, pm_b2, pm_g, pm_beta,
           pn_W1, pn_b1, pn_W2, pn_b2, pn_g, pn_beta, dec_W1, dec_b1,
           dec_W2, dec_b2):
    f32 = jnp.float32
    r2 = lambda v: v.reshape(1, -1)
    src = edge_index[0]
    dst = edge_index[1]

    x_h = _call(
        _enc_node_body, N // _BN,
        [_row_spec(_BN, 8), _const_spec((1, 1)),
         _const_spec((8, H)), _const_spec((1, H)), _const_spec((1, H)),
         _const_spec((H, H)), _const_spec((1, H)), _const_spec((1, H)),
         _const_spec((1, H))],
        _row_spec(_BN, H), jax.ShapeDtypeStruct((N, H), f32),
    )(x, swelling_phi.reshape(1, 1), enc_n_W1[:8], r2(enc_n_W1[8]),
      r2(enc_n_b1), enc_n_W2, r2(enc_n_b2), r2(enc_n_g), r2(enc_n_beta))

    e_h = _call(
        _enc_edge_body, E // _BE,
        [_row_spec(_BE, 4), _const_spec((4, H)), _const_spec((1, H)),
         _const_spec((H, H)), _const_spec((1, H)), _const_spec((1, H)),
         _const_spec((1, H))],
        _row_spec(_BE, H), jax.ShapeDtypeStruct((E, H), f32),
    )(edge_attr, enc_e_W1, r2(enc_e_b1), enc_e_W2, r2(enc_e_b2),
      r2(enc_e_g), r2(enc_e_beta))

    for i in range(3):
        x_i = jnp.take(x_h, src, axis=0)
        x_j = jnp.take(x_h, dst, axis=0)

        e_h, msg = _call(
            _edge_body, E // _BE,
            [_row_spec(_BE, H)] * 3 +
            [_const_spec((H, H))] * 3 +
            [_const_spec((1, H)), _const_spec((H, H)), _const_spec((1, H)),
             _const_spec((1, H)), _const_spec((1, H))] +
            [_const_spec((H, H))] * 2 +
            [_const_spec((1, H)), _const_spec((H, H)), _const_spec((1, H)),
             _const_spec((1, H)), _const_spec((1, H))],
            [_row_spec(_BE, H), _row_spec(_BE, H)],
            [jax.ShapeDtypeStruct((E, H), f32),
             jax.ShapeDtypeStruct((E, H), f32)],
        )(x_i, x_j, e_h,
          pe_W1[i, 0:H], pe_W1[i, H:2 * H], pe_W1[i, 2 * H:3 * H],
          r2(pe_b1[i]), pe_W2[i], r2(pe_b2[i]), r2(pe_g[i]), r2(pe_beta[i]),
          pm_W1[i, 0:H], pm_W1[i, H:2 * H],
          r2(pm_b1[i]), pm_W2[i], r2(pm_b2[i]), r2(pm_g[i]), r2(pm_beta[i]))

        agg = jax.ops.segment_sum(msg, dst, num_segments=N)

        x_h = _call(
            _node_body, N // _BN,
            [_row_spec(_BN, H), _row_spec(_BN, H),
             _const_spec((H, H)), _const_spec((H, H)), _const_spec((1, H)),
             _const_spec((H, H)), _const_spec((1, H)), _const_spec((1, H)),
             _const_spec((1, H))],
            _row_spec(_BN, H), jax.ShapeDtypeStruct((N, H), f32),
        )(x_h, agg, pn_W1[i, 0:H], pn_W1[i, H:2 * H], r2(pn_b1[i]),
          pn_W2[i], r2(pn_b2[i]), r2(pn_g[i]), r2(pn_beta[i]))

    out = _call(
        _dec_body, N // _BN,
        [_row_spec(_BN, H), _const_spec((H, H)), _const_spec((1, H)),
         _const_spec((H, 3)), _const_spec((1, 3))],
        _row_spec(_BN, 3), jax.ShapeDtypeStruct((N, 3), f32),
    )(x_h, dec_W1, r2(dec_b1), dec_W2, r2(dec_b2))
    return out


# speedup vs baseline: 1.5982x; 1.5982x over previous
import functools

import jax
import jax.numpy as jnp
from jax import lax
from jax.experimental import pallas as pl
from jax.experimental.pallas import tpu as pltpu
from jax.experimental.pallas import tpu_sc as plsc

N = 50000
E = 800000
H = 64
H2 = 128

_BN = 2000
_BE = 4000

_EPS = 1e-5

_NC, _NS = 2, 16
_NW = _NC * _NS
_GK = 40
_EW = E // _NW
_NCH = _EW // _GK
_GNB = 5
_SK = 80
_ES = E // _NS
_SCH = _ES // _SK
_SNB = 2
_FQ = 16
_ACC = 51200
_ZR = 80
_WR = 200


def _ln(h, g, beta):
    mu = jnp.mean(h, axis=-1, keepdims=True)
    var = jnp.mean((h - mu) ** 2, axis=-1, keepdims=True)
    return (h - mu) * lax.rsqrt(var + _EPS) * g + beta


def _dot(a, b):
    return jax.lax.dot_general(a, b, (((1,), (0,)), ((), ())),
                               preferred_element_type=jnp.float32)


def _gather_body(t1, t2, src, dst, out_i, out_j,
                 sidx, didx, st_a, st_b, *sems):
    gsa = sems[:_GNB]
    gsb = sems[_GNB:2 * _GNB]
    wsem = sems[2 * _GNB]
    c = lax.axis_index("c")
    s = lax.axis_index("s")
    wid = s * _NC + c
    base = wid * _EW
    pltpu.sync_copy(src.at[pl.ds(base, _EW)], sidx)
    pltpu.sync_copy(dst.at[pl.ds(base, _EW)], didx)

    def outer(t, carry):
        j0 = t * _GNB
        hs = []
        for b in range(_GNB):
            e0 = (j0 + b) * _GK
            hs.append(pltpu.async_copy(t1.at[sidx.at[pl.ds(e0, _GK)]],
                                       st_a.at[b], gsa[b]))
            hs.append(pltpu.async_copy(t2.at[didx.at[pl.ds(e0, _GK)]],
                                       st_b.at[b], gsb[b]))
        ws = []
        for b in range(_GNB):
            row = base + (j0 + b) * _GK
            hs[2 * b].wait()
            ws.append(pltpu.async_copy(st_a.at[b],
                                       out_i.at[pl.ds(row, _GK)], wsem))
            hs[2 * b + 1].wait()
            ws.append(pltpu.async_copy(st_b.at[b],
                                       out_j.at[pl.ds(row, _GK)], wsem))
        for w in ws:
            w.wait()
        return carry

    lax.fori_loop(0, _NCH // _GNB, outer, 0)


@functools.partial(
    pl.kernel,
    mesh=plsc.VectorSubcoreMesh(core_axis_name="c", subcore_axis_name="s"),
    out_type=[jax.ShapeDtypeStruct((E, H2), jnp.float32),
              jax.ShapeDtypeStruct((E, H2), jnp.float32)],
    scratch_types=[pltpu.VMEM((_EW,), jnp.int32),
                   pltpu.VMEM((_EW,), jnp.int32),
                   pltpu.VMEM((_GNB, _GK, H2), jnp.float32),
                   pltpu.VMEM((_GNB, _GK, H2), jnp.float32)]
                  + [pltpu.SemaphoreType.DMA] * (2 * _GNB + 1),
)
def _sc_gather(t1, t2, src, dst, out_i, out_j, *rest):
    _gather_body(t1, t2, src, dst, out_i, out_j, *rest)


def _scatter_body(msg4, tdst, out, didx, stg, zb, acc, *sems):
    c = lax.axis_index("c")
    s = lax.axis_index("s")

    z16 = jnp.zeros((16,), jnp.float32)
    for r in range(_ZR):
        zb[r, pl.ds(0, _FQ)] = z16

    for p in range(2):
        q = 2 * p + c

        for r in range(_ACC // _ZR // _NS):
            row = (s * (_ACC // _NS)) + r * _ZR
            pltpu.sync_copy(zb, acc.at[pl.ds(row, _ZR)])
        plsc.subcore_barrier()

        def outer(t, carry):
            j0 = t * _SNB
            hs = []
            for b in range(_SNB):
                hs.append(pltpu.async_copy(msg4.at[q, s * _SCH + j0 + b],
                                           stg.at[b], sems[b]))
                hs.append(pltpu.async_copy(tdst.at[s, j0 + b],
                                           didx.at[b], sems[_SNB + b]))
            for b in range(_SNB):
                hs[2 * b].wait()
                hs[2 * b + 1].wait()
                pltpu.sync_copy(stg.at[b], acc.at[didx.at[b]],
                                add=True)
            return carry

        lax.fori_loop(0, _SCH // _SNB, outer, 0)

        plsc.subcore_barrier()
        nw = N // _WR

        def wloop(t, carry):
            r = t * _NS + s

            @pl.when(r < nw)
            def _():
                pltpu.sync_copy(acc.at[pl.ds(r * _WR, _WR)],
                                out.at[q, pl.ds(r * _WR, _WR)])
            return carry

        lax.fori_loop(0, (nw + _NS - 1) // _NS, wloop, 0)
        plsc.subcore_barrier()


@functools.partial(
    pl.kernel,
    mesh=plsc.VectorSubcoreMesh(core_axis_name="c", subcore_axis_name="s"),
    out_type=jax.ShapeDtypeStruct((4, N, _FQ), jnp.float32),
    scratch_types=[pltpu.VMEM((_SNB, _SK), jnp.int32),
                   pltpu.VMEM((_SNB, _SK, _FQ), jnp.float32),
                   pltpu.VMEM((_ZR, _FQ), jnp.float32),
                   pltpu.VMEM_SHARED((_ACC, _FQ), jnp.float32)]
                  + [pltpu.SemaphoreType.DMA] * (2 * _SNB),
)
def _sc_scatter(msg4, tdst, out, *rest):
    _scatter_body(msg4, tdst, out, *rest)


def _enc_node_body(x_ref, phi_ref, w1a_ref, w1b_ref, b1_ref, w2_ref, b2_ref,
                   g_ref, beta_ref, out_ref):
    x = x_ref[...]
    phi = phi_ref[0, 0]
    load = jnp.where(x[:, 7:8] == 1.0, phi, 0.0)
    h = _dot(x, w1a_ref[...]) + load * w1b_ref[...] + b1_ref[...]
    h = jax.nn.relu(h)
    h = _dot(h, w2_ref[...]) + b2_ref[...]
    out_ref[...] = _ln(h, g_ref[...], beta_ref[...])


def _enc_edge_body(a_ref, w1_ref, b1_ref, w2_ref, b2_ref, g_ref, beta_ref,
                   out_ref):
    h = _dot(a_ref[...], w1_ref[...]) + b1_ref[...]
    h = jax.nn.relu(h)
    h = _dot(h, w2_ref[...]) + b2_ref[...]
    out_ref[...] = _ln(h, g_ref[...], beta_ref[...])


def _pre_body(xh_ref, a1_ref, m1_ref, a2_ref, t1_ref, t2_ref):
    xh = xh_ref[...]
    t1_ref[:, :H] = _dot(xh, a1_ref[...])
    t1_ref[:, H:] = _dot(xh, m1_ref[...])
    t2_ref[:, :H] = _dot(xh, a2_ref[...])
    t2_ref[:, H:] = jnp.zeros_like(xh)


def _edge_body(g1_ref, g2_ref, eh_ref,
               a3_ref, eb1_ref, ew2_ref, eb2_ref, eg_ref, ebeta_ref,
               m2_ref, mb1_ref, mw2_ref, mb2_ref, mg_ref, mbeta_ref,
               eout_ref, msg_ref):
    g1 = g1_ref[...]
    eh = eh_ref[...]
    pij = g1[:, :H] + g2_ref[:, :H]
    h = jax.nn.relu(pij + _dot(eh, a3_ref[...]) + eb1_ref[...])
    e_upd = _ln(_dot(h, ew2_ref[...]) + eb2_ref[...], eg_ref[...],
                ebeta_ref[...])
    h2 = jax.nn.relu(g1[:, H:] + _dot(e_upd, m2_ref[...]) + mb1_ref[...])
    msg = _ln(_dot(h2, mw2_ref[...]) + mb2_ref[...], mg_ref[...],
              mbeta_ref[...])
    eout_ref[...] = e_upd + eh
    for qq in range(4):
        msg_ref[qq] = msg[:, qq * _FQ:(qq + 1) * _FQ]


def _node_body(xh_ref, a0_ref, a1_ref, a2_ref, a3_ref, n1_ref, n2_ref,
               b1_ref, w2_ref, b2_ref, g_ref, beta_ref, out_ref):
    xh = xh_ref[...]
    agg = jnp.concatenate(
        [a0_ref[...], a1_ref[...], a2_ref[...], a3_ref[...]], axis=1)
    h = jax.nn.relu(_dot(xh, n1_ref[...]) + _dot(agg, n2_ref[...])
                    + b1_ref[...])
    h = _dot(h, w2_ref[...]) + b2_ref[...]
    out_ref[...] = _ln(h, g_ref[...], beta_ref[...]) + xh


def _dec_body(xh_ref, w1_ref, b1_ref, w2_ref, b2_ref, out_ref):
    h = jax.nn.relu(_dot(xh_ref[...], w1_ref[...]) + b1_ref[...])
    out_ref[...] = _dot(h, w2_ref[...]) + b2_ref[...]


def _row_spec(bs, width):
    return pl.BlockSpec((bs, width), lambda i: (i, 0))


def _const_spec(shape):
    nd = len(shape)
    return pl.BlockSpec(shape, lambda i: (0,) * nd)


def _call(body, grid, in_specs, out_specs, out_shape):
    return pl.pallas_call(
        body,
        grid=(grid,),
        in_specs=in_specs,
        out_specs=out_specs,
        out_shape=out_shape,
        compiler_params=pltpu.CompilerParams(
            dimension_semantics=("arbitrary",)),
    )


def kernel(x, edge_index, edge_attr, swelling_phi, enc_n_W1, enc_n_b1,
           enc_n_W2, enc_n_b2, enc_n_g, enc_n_beta, enc_e_W1, enc_e_b1,
           enc_e_W2, enc_e_b2, enc_e_g, enc_e_beta, pe_W1, pe_b1, pe_W2,
           pe_b2, pe_g, pe_beta, pm_W1, pm_b1, pm_W2, pm_b2, pm_g, pm_beta,
           pn_W1, pn_b1, pn_W2, pn_b2, pn_g, pn_beta, dec_W1, dec_b1,
           dec_W2, dec_b2):
    f32 = jnp.float32
    r2 = lambda v: v.reshape(1, -1)
    src = edge_index[0]
    dst = edge_index[1]
    tdst = dst.reshape(_NS, _SCH, _SK)
    msg4_shape = (4, E // _SK, _SK, _FQ)

    x_h = _call(
        _enc_node_body, N // _BN,
        [_row_spec(_BN, 8), _const_spec((1, 1)),
         _const_spec((8, H)), _const_spec((1, H)), _const_spec((1, H)),
         _const_spec((H, H)), _const_spec((1, H)), _const_spec((1, H)),
         _const_spec((1, H))],
        _row_spec(_BN, H), jax.ShapeDtypeStruct((N, H), f32),
    )(x, swelling_phi.reshape(1, 1), enc_n_W1[:8], r2(enc_n_W1[8]),
      r2(enc_n_b1), enc_n_W2, r2(enc_n_b2), r2(enc_n_g), r2(enc_n_beta))

    e_h = _call(
        _enc_edge_body, E // _BE,
        [_row_spec(_BE, 4), _const_spec((4, H)), _const_spec((1, H)),
         _const_spec((H, H)), _const_spec((1, H)), _const_spec((1, H)),
         _const_spec((1, H))],
        _row_spec(_BE, H), jax.ShapeDtypeStruct((E, H), f32),
    )(edge_attr, enc_e_W1, r2(enc_e_b1), enc_e_W2, r2(enc_e_b2),
      r2(enc_e_g), r2(enc_e_beta))

    for i in range(3):
        t1, t2 = _call(
            _pre_body, N // _BN,
            [_row_spec(_BN, H)] + [_const_spec((H, H))] * 3,
            [_row_spec(_BN, H2), _row_spec(_BN, H2)],
            [jax.ShapeDtypeStruct((N, H2), f32),
             jax.ShapeDtypeStruct((N, H2), f32)],
        )(x_h, pe_W1[i, 0:H], pm_W1[i, 0:H], pe_W1[i, H:2 * H])

        g1, g2 = _sc_gather(t1, t2, src, dst)

        e_h, msg = _call(
            _edge_body, E // _BE,
            [_row_spec(_BE, H2), _row_spec(_BE, H2), _row_spec(_BE, H),
             _const_spec((H, H)), _const_spec((1, H)), _const_spec((H, H)),
             _const_spec((1, H)), _const_spec((1, H)), _const_spec((1, H)),
             _const_spec((H, H)), _const_spec((1, H)), _const_spec((H, H)),
             _const_spec((1, H)), _const_spec((1, H)), _const_spec((1, H))],
            [_row_spec(_BE, H),
             pl.BlockSpec((4, _BE, _FQ), lambda i: (0, i, 0))],
            [jax.ShapeDtypeStruct((E, H), f32),
             jax.ShapeDtypeStruct((4, E, _FQ), f32)],
        )(g1, g2, e_h,
          pe_W1[i, 2 * H:3 * H], r2(pe_b1[i]), pe_W2[i], r2(pe_b2[i]),
          r2(pe_g[i]), r2(pe_beta[i]),
          pm_W1[i, H:2 * H], r2(pm_b1[i]), pm_W2[i], r2(pm_b2[i]),
          r2(pm_g[i]), r2(pm_beta[i]))

        _m = jnp.moveaxis(msg, 0, 1).reshape(E, H)
        _a = jax.ops.segment_sum(_m, dst, num_segments=N)
        agg = jnp.moveaxis(_a.reshape(N, 4, _FQ), 1, 0)

        x_h = _call(
            _node_body, N // _BN,
            [_row_spec(_BN, H)] + [_row_spec(_BN, _FQ)] * 4 +
            [_const_spec((H, H)), _const_spec((H, H)), _const_spec((1, H)),
             _const_spec((H, H)), _const_spec((1, H)), _const_spec((1, H)),
             _const_spec((1, H))],
            _row_spec(_BN, H), jax.ShapeDtypeStruct((N, H), f32),
        )(x_h, agg[0], agg[1], agg[2], agg[3],
          pn_W1[i, 0:H], pn_W1[i, H:2 * H], r2(pn_b1[i]),
          pn_W2[i], r2(pn_b2[i]), r2(pn_g[i]), r2(pn_beta[i]))

    out = _call(
        _dec_body, N // _BN,
        [_row_spec(_BN, H), _const_spec((H, H)), _const_spec((1, H)),
         _const_spec((H, 3)), _const_spec((1, 3))],
        _row_spec(_BN, 3), jax.ShapeDtypeStruct((N, 3), f32),
    )(x_h, dec_W1, r2(dec_b1), dec_W2, r2(dec_b2))
    return out
